# Initial kernel scaffold; baseline (speedup 1.0000x reference)
import jax
import jax.numpy as jnp
from jax.experimental import pallas as pl


def kernel(node_features, edge_index_input, edge_prob_input, x, W_proj, W_prob, a_src, a_trg, a_prob, W_skip, gat_bias, in_proj_W, conv_w, conv_b, x_proj_W, dt_proj_W, dt_proj_b, A_log, D_param, mamba_out_W, ln_g, ln_b, W_out, b_out):
    raise NotImplementedError("write your pallas kernel here")


# speedup vs baseline: 9.9357x; 9.9357x over previous
import functools

import jax
import jax.numpy as jnp
from jax import lax
from jax.experimental import pallas as pl
from jax.experimental.pallas import tpu as pltpu
from jax.experimental.pallas import tpu_sc as plsc

N = 10000
NPAD = 10240
E = 320000
NH = 8
F = 32
DM = 256
DI = 256
DS = 8
DTR = 16
BSEQ = 16
TSEQ = 512

NC = 2
NS = 16
NW = NC * NS
RPT = NPAD // NS

BN = 256
NB = NPAD // BN

CA = 80
EW_A = E // NW
CB = 80
EW_B = E // NS

_f32 = jnp.float32
_mesh = plsc.VectorSubcoreMesh(core_axis_name="c", subcore_axis_name="s")


def _k1_body(nf_ref, wp_ref, wsk_ref, asrc_ref, atrg_ref, wprob_ref, aprob_ref,
             proj2_ref, skip2_ref, ss_ref, st_ref, cv_ref):
    nf = nf_ref[...]
    proj = jnp.dot(nf, wp_ref[...], preferred_element_type=_f32)
    proj2_ref[0] = proj
    skip2_ref[0] = jnp.dot(nf, wsk_ref[...], preferred_element_type=_f32)
    p3 = proj.reshape(BN, 4, 32)
    ss_ref[0] = (p3 * asrc_ref[0]).sum(-1)
    st_ref[0] = (p3 * atrg_ref[0]).sum(-1)
    cv = (wprob_ref[0] * aprob_ref[0]).sum(-1)
    cv_ref[0] = jnp.broadcast_to(cv[None, :], (8, 4))


_k1 = pl.pallas_call(
    _k1_body,
    grid=(2, NB),
    in_specs=[
        pl.BlockSpec((BN, 128), lambda j, i: (i, 0)),
        pl.BlockSpec((128, 128), lambda j, i: (0, j)),
        pl.BlockSpec((128, 128), lambda j, i: (0, j)),
        pl.BlockSpec((1, 4, 32), lambda j, i: (j, 0, 0)),
        pl.BlockSpec((1, 4, 32), lambda j, i: (j, 0, 0)),
        pl.BlockSpec((1, 4, 32), lambda j, i: (j, 0, 0)),
        pl.BlockSpec((1, 4, 32), lambda j, i: (j, 0, 0)),
    ],
    out_specs=[
        pl.BlockSpec((1, BN, 128), lambda j, i: (j, i, 0)),
        pl.BlockSpec((1, BN, 128), lambda j, i: (j, i, 0)),
        pl.BlockSpec((1, BN, 4), lambda j, i: (j, i, 0)),
        pl.BlockSpec((1, BN, 4), lambda j, i: (j, i, 0)),
        pl.BlockSpec((1, 8, 4), lambda j, i: (j, 0, 0)),
    ],
    out_shape=[
        jax.ShapeDtypeStruct((2, NPAD, 128), _f32),
        jax.ShapeDtypeStruct((2, NPAD, 128), _f32),
        jax.ShapeDtypeStruct((2, NPAD, 4), _f32),
        jax.ShapeDtypeStruct((2, NPAD, 4), _f32),
        jax.ShapeDtypeStruct((2, 8, 4), _f32),
    ],
)


def _make_ka():

    def body(sstbl_s, sstbl_t, srcl, trgl, epl, cvec, w_out, dpart,
             srcidx_v, trgidx_v, ep_v, cv_v, srows_v, trows_v, wbuf_v,
             dsp, sem):
        c = lax.axis_index("c")
        s = lax.axis_index("s")
        wid = s * NC + c

        def _zr(i, _):
            wbuf_v[i] = jnp.zeros((16,), _f32)
            return 0
        lax.fori_loop(0, CA, _zr, 0)
        for q in range(RPT // CA):
            pltpu.sync_copy(wbuf_v, dsp.at[pl.ds(s * RPT + q * CA, CA)])
        plsc.subcore_barrier()

        pltpu.sync_copy(cvec, cv_v)
        cv = cv_v[...]
        iot = lax.iota(jnp.int32, 16)
        lane8 = iot < 8

        def _chunk(k, _):
            base = wid * EW_A + k * CA
            pltpu.sync_copy(srcl.at[pl.ds(base, CA)], srcidx_v)
            pltpu.sync_copy(trgl.at[pl.ds(base, CA)], trgidx_v)
            pltpu.sync_copy(epl.at[pl.ds(base, CA)], ep_v.at[pl.ds(0, CA)])
            pltpu.async_copy(sstbl_s.at[srcidx_v], srows_v, sem).wait()
            pltpu.async_copy(sstbl_t.at[trgidx_v], trows_v, sem).wait()

            def _edge(e, _):
                pe = ep_v[pl.ds(e, 16)][0]
                sc = (srows_v[e, pl.ds(0, 16)] + trows_v[e, pl.ds(0, 16)]
                      + pe * cv)
                sc = jnp.maximum(sc, 0.2 * sc)
                w = jnp.where(lane8, jnp.exp(sc), 0.0)
                wbuf_v[e] = w
                return 0
            lax.fori_loop(0, CA, _edge, 0)
            pltpu.sync_copy(wbuf_v, w_out.at[pl.ds(base, CA)])
            pltpu.sync_copy(wbuf_v, dsp.at[trgidx_v], add=True)
            return 0
        lax.fori_loop(0, EW_A // CA, _chunk, 0)

        plsc.subcore_barrier()
        for q in range(RPT // CA):
            r0 = s * RPT + q * CA
            pltpu.sync_copy(dsp.at[pl.ds(r0, CA)],
                            dpart.at[pl.ds(c * NPAD + r0, CA)])

    return pl.kernel(
        body,
        out_type=[
            jax.ShapeDtypeStruct((E, 16), _f32),
            jax.ShapeDtypeStruct((NC * NPAD, 16), _f32),
        ],
        mesh=_mesh,
        scratch_types=[
            pltpu.VMEM((CA,), jnp.int32),
            pltpu.VMEM((CA,), jnp.int32),
            pltpu.VMEM((CA + 16,), _f32),
            pltpu.VMEM((16,), _f32),
            pltpu.VMEM((CA, 128), _f32),
            pltpu.VMEM((CA, 128), _f32),
            pltpu.VMEM((CA, 16), _f32),
            pltpu.VMEM_SHARED((NPAD, 16), _f32),
            pltpu.SemaphoreType.DMA,
        ],
    )


_ka = _make_ka()


def _make_spmm(mode):
    def body(table, gidxl, sidxl, scalel, out2,
             gidx_v, sidx_v, w_v, rows_v, acc_sp, sem):
        c = lax.axis_index("c")
        s = lax.axis_index("s")

        def _zr(i, _):
            for k2 in range(8):
                rows_v[i, pl.ds(16 * k2, 16)] = jnp.zeros((16,), _f32)
            return 0
        lax.fori_loop(0, CB, _zr, 0)
        for q in range(RPT // CB):
            pltpu.sync_copy(rows_v, acc_sp.at[pl.ds(s * RPT + q * CB, CB)])
        plsc.subcore_barrier()

        coff = c * NPAD

        def _chunk(k, _):
            base = s * EW_B + k * CB
            pltpu.sync_copy(gidxl.at[pl.ds(base, CB)], gidx_v)
            pltpu.sync_copy(sidxl.at[pl.ds(base, CB)], sidx_v)
            if mode == "gat":
                pltpu.sync_copy(scalel.at[pl.ds(base, CB)], w_v)
            else:
                pltpu.sync_copy(scalel.at[pl.ds(base, CB)], w_v.at[pl.ds(0, CB)])
            for i in range(CB // 16):
                gidx_v[pl.ds(16 * i, 16)] = gidx_v[pl.ds(16 * i, 16)] + coff
            pltpu.async_copy(table.at[gidx_v], rows_v, sem).wait()

            if mode == "gat":
                lbase = 4 * c

                def _edge(e, _):
                    wrow = w_v[e]
                    for m in range(4):
                        idxv = jnp.zeros((16,), jnp.int32) + (lbase + m)
                        b = wrow.at[idxv].get(mode="promise_in_bounds")
                        for k2 in (2 * m, 2 * m + 1):
                            sl = pl.ds(16 * k2, 16)
                            rows_v[e, sl] = rows_v[e, sl] * b
                    return 0
            else:

                def _edge(e, _):
                    pe = w_v[pl.ds(e, 16)][0]
                    for k2 in range(8):
                        sl = pl.ds(16 * k2, 16)
                        rows_v[e, sl] = rows_v[e, sl] * pe
                    return 0
            lax.fori_loop(0, CB, _edge, 0)
            pltpu.sync_copy(rows_v, acc_sp.at[sidx_v], add=True)
            return 0
        lax.fori_loop(0, EW_B // CB, _chunk, 0)

        plsc.subcore_barrier()
        for q in range(RPT // CB):
            r0 = s * RPT + q * CB
            pltpu.sync_copy(acc_sp.at[pl.ds(r0, CB)],
                            out2.at[pl.ds(coff + r0, CB)])

    scale_scratch = (pltpu.VMEM((CB, 16), _f32) if mode == "gat"
                     else pltpu.VMEM((CB + 16,), _f32))
    return pl.kernel(
        body,
        out_type=jax.ShapeDtypeStruct((NC * NPAD, 128), _f32),
        mesh=_mesh,
        scratch_types=[
            pltpu.VMEM((CB,), jnp.int32),
            pltpu.VMEM((CB,), jnp.int32),
            scale_scratch,
            pltpu.VMEM((CB, 128), _f32),
            pltpu.VMEM_SHARED((NPAD, 128), _f32),
            pltpu.SemaphoreType.DMA,
        ],
    )


_kb_gat = _make_spmm("gat")
_kb_hop = _make_spmm("hop")


def _k4_body(num_ref, d0_ref, d1_ref, skip_ref, bias_ref, x0_ref):
    j = pl.program_id(0)
    d = d0_ref[0] + d1_ref[0]
    d4 = jnp.where(j == 0, d[:, 0:4], d[:, 4:8])
    d128 = jnp.broadcast_to(d4[:, :, None], (BN, 4, 32)).reshape(BN, 128)
    o = num_ref[0] / (d128 + 1e-16) + skip_ref[0] + bias_ref[0]
    x0_ref[0] = jnp.where(o > 0, o, jnp.exp(jnp.minimum(o, 0.0)) - 1.0)


_k4 = pl.pallas_call(
    _k4_body,
    grid=(2, NB),
    in_specs=[
        pl.BlockSpec((1, BN, 128), lambda j, i: (j, i, 0)),
        pl.BlockSpec((1, BN, 16), lambda j, i: (0, i, 0)),
        pl.BlockSpec((1, BN, 16), lambda j, i: (1, i, 0)),
        pl.BlockSpec((1, BN, 128), lambda j, i: (j, i, 0)),
        pl.BlockSpec((1, 1, 128), lambda j, i: (j, 0, 0)),
    ],
    out_specs=pl.BlockSpec((1, BN, 128), lambda j, i: (j, i, 0)),
    out_shape=jax.ShapeDtypeStruct((2, NPAD, 128), _f32),
)


def _sig(v):
    return 1.0 / (1.0 + jnp.exp(-v))


def _k6_body(x00, x01, x10, x11, x20, x21, x30, x31,
             ipw_ref, cwt_ref, xpw_ref, dtw_ref, alt_ref, mow_ref, wout_ref,
             vecs_ref, emb_ref):
    xs = [jnp.concatenate([a[0], b[0]], axis=1)
          for a, b in ((x00, x01), (x10, x11), (x20, x21), (x30, x31))]
    conv_b = vecs_ref[0:1]
    dt_b = vecs_ref[1:2]
    d_param = vecs_ref[2:3]
    ln_g = vecs_ref[3:4]
    ln_b = vecs_ref[4:5]
    b_out = vecs_ref[5:6]

    xm = []
    z3 = None
    for t in range(4):
        xz = jnp.dot(xs[t], ipw_ref[...], preferred_element_type=_f32)
        xm.append(xz[:, :DI])
        if t == 3:
            z3 = xz[:, DI:]
    cwt = cwt_ref[...]
    xmc = []
    for t in range(4):
        acc = jnp.broadcast_to(conv_b, (BN, DI))
        for jj in range(t + 1):
            acc = acc + xm[jj] * cwt[jj + 3 - t]
        xmc.append(acc * _sig(acc))
    A = -jnp.exp(alt_ref[...])
    hs = [jnp.zeros((BN, DI), _f32) for _ in range(DS)]
    y3 = None
    for t in range(4):
        xd = jnp.dot(xmc[t], xpw_ref[...], preferred_element_type=_f32)
        dtv = jnp.dot(xd[:, :DTR], dtw_ref[...],
                      preferred_element_type=_f32) + dt_b
        dtv = jnp.maximum(dtv, 0.0) + jnp.log(1.0 + jnp.exp(-jnp.abs(dtv)))
        u = dtv * xmc[t]
        Bm = xd[:, DTR:DTR + DS]
        for s2 in range(DS):
            hs[s2] = hs[s2] * jnp.exp(dtv * A[s2]) + u * Bm[:, s2:s2 + 1]
        if t == 3:
            C3 = xd[:, DTR + DS:DTR + 2 * DS]
            y3 = hs[0] * C3[:, 0:1]
            for s2 in range(1, DS):
                y3 = y3 + hs[s2] * C3[:, s2:s2 + 1]
    y = y3 + xmc[3] * d_param
    y = y * (z3 * _sig(z3))
    o3 = jnp.dot(y, mow_ref[...], preferred_element_type=_f32)
    v = o3 + xs[0]
    mu = jnp.mean(v, axis=1, keepdims=True)
    var = jnp.mean((v - mu) ** 2, axis=1, keepdims=True)
    lnv = (v - mu) / jnp.sqrt(var + 1e-5) * ln_g + ln_b
    emb_ref[...] = jnp.dot(lnv, wout_ref[...], preferred_element_type=_f32) + b_out


def _xspec(h):
    return pl.BlockSpec((1, BN, 128), lambda i, _h=h: (_h, i, 0))


_k6 = pl.pallas_call(
    _k6_body,
    grid=(NB,),
    in_specs=(
        [_xspec(h) for _t in range(4) for h in (0, 1)]
        + [
            pl.BlockSpec((DM, 2 * DI), lambda i: (0, 0)),
            pl.BlockSpec((4, 1, DI), lambda i: (0, 0, 0)),
            pl.BlockSpec((DI, DTR + 2 * DS), lambda i: (0, 0)),
            pl.BlockSpec((DTR, DI), lambda i: (0, 0)),
            pl.BlockSpec((DS, DI), lambda i: (0, 0)),
            pl.BlockSpec((DI, DM), lambda i: (0, 0)),
            pl.BlockSpec((DM, DM), lambda i: (0, 0)),
            pl.BlockSpec((8, DM), lambda i: (0, 0)),
        ]
    ),
    out_specs=pl.BlockSpec((BN, DM), lambda i: (i, 0)),
    out_shape=jax.ShapeDtypeStruct((NPAD, DM), _f32),
)


def _k7_body(emb, idxl, out, idx_v, rows_v, sem):
    c = lax.axis_index("c")
    s = lax.axis_index("s")
    wid = s * NC + c
    for q in range(2):
        base = wid * (BSEQ * TSEQ // NW) + q * 128
        pltpu.sync_copy(idxl.at[pl.ds(base, 128)], idx_v)
        pltpu.async_copy(emb.at[idx_v], rows_v, sem).wait()
        pltpu.sync_copy(rows_v, out.at[pl.ds(base, 128)])


_k7 = pl.kernel(
    _k7_body,
    out_type=jax.ShapeDtypeStruct((BSEQ * TSEQ, DM), _f32),
    mesh=_mesh,
    scratch_types=[
        pltpu.VMEM((128,), jnp.int32),
        pltpu.VMEM((128, DM), _f32),
        pltpu.SemaphoreType.DMA,
    ],
)


def kernel(node_features, edge_index_input, edge_prob_input, x, W_proj, W_prob,
           a_src, a_trg, a_prob, W_skip, gat_bias, in_proj_W, conv_w, conv_b,
           x_proj_W, dt_proj_W, dt_proj_b, A_log, D_param, mamba_out_W,
           ln_g, ln_b, W_out, b_out):
    nfp = jnp.pad(node_features, ((0, NPAD - N), (0, 0)))
    src = edge_index_input[0]
    trg = edge_index_input[1]
    ep = edge_prob_input.reshape(E)

    proj2, skip2, ss_o, st_o, cv_o = _k1(
        nfp, W_proj, W_skip,
        a_src.reshape(2, 4, 32), a_trg.reshape(2, 4, 32),
        W_prob.reshape(2, 4, 32), a_prob.reshape(2, 4, 32))

    z120 = jnp.zeros((NPAD, 120), _f32)
    sstbl_s = jnp.concatenate([ss_o[0], ss_o[1], z120], axis=1)
    sstbl_t = jnp.concatenate([st_o[0], st_o[1], z120], axis=1)
    cvec16 = jnp.concatenate([cv_o[0, 0], cv_o[1, 0], jnp.zeros((8,), _f32)])

    w_e, dpart = _ka(sstbl_s, sstbl_t, src, trg, ep, cvec16)

    num2 = _kb_gat(proj2.reshape(NC * NPAD, 128), src, trg, w_e)

    x0_2 = _k4(num2.reshape(2, NPAD, 128), dpart.reshape(2, NPAD, 16),
               dpart.reshape(2, NPAD, 16), skip2, gat_bias.reshape(2, 1, 128))

    x0f = x0_2.reshape(NC * NPAD, 128)
    x1f = _kb_hop(x0f, trg, src, ep)
    x2f = _kb_hop(x1f, trg, src, ep)
    x3f = _kb_hop(x2f, trg, src, ep)

    vecs8 = jnp.stack([conv_b, dt_proj_b, D_param, ln_g, ln_b, b_out,
                       jnp.zeros((DM,), _f32), jnp.zeros((DM,), _f32)])
    xr = [v.reshape(2, NPAD, 128) for v in (x0f, x1f, x2f, x3f)]
    emb = _k6(xr[0], xr[0], xr[1], xr[1], xr[2], xr[2], xr[3], xr[3],
              in_proj_W, conv_w.T.reshape(4, 1, DI), x_proj_W, dt_proj_W,
              A_log.T, mamba_out_W, W_out, vecs8)

    out = _k7(emb, x.reshape(-1))
    return out.reshape(BSEQ, TSEQ, DM)


# speedup vs baseline: 13.7682x; 1.3857x over previous
import functools

import jax
import jax.numpy as jnp
from jax import lax
from jax.experimental import pallas as pl
from jax.experimental.pallas import tpu as pltpu
from jax.experimental.pallas import tpu_sc as plsc

N = 10000
NPAD = 10240
E = 320000
NH = 8
F = 32
DM = 256
DI = 256
DS = 8
DTR = 16
BSEQ = 16
TSEQ = 512

NC = 2
NS = 16
NW = NC * NS
RPT = NPAD // NS

BN = 256
NB = NPAD // BN

CA = 80
EW_A = E // NW
CB = 80
EW_B = E // NS

_f32 = jnp.float32
_mesh = plsc.VectorSubcoreMesh(core_axis_name="c", subcore_axis_name="s")


def _k1_body(nf_ref, wp_ref, wsk_ref, asrc_ref, atrg_ref, wprob_ref, aprob_ref,
             proj2_ref, skip2_ref, ss_ref, st_ref, cv_ref):
    nf = nf_ref[...]
    proj = jnp.dot(nf, wp_ref[...], preferred_element_type=_f32)
    proj2_ref[0] = proj
    skip2_ref[0] = jnp.dot(nf, wsk_ref[...], preferred_element_type=_f32)
    p3 = proj.reshape(BN, 4, 32)
    ss_ref[0] = (p3 * asrc_ref[0]).sum(-1)
    st_ref[0] = (p3 * atrg_ref[0]).sum(-1)
    cv = (wprob_ref[0] * aprob_ref[0]).sum(-1)
    cv_ref[0] = jnp.broadcast_to(cv[None, :], (8, 4))


_k1 = pl.pallas_call(
    _k1_body,
    grid=(2, NB),
    in_specs=[
        pl.BlockSpec((BN, 128), lambda j, i: (i, 0)),
        pl.BlockSpec((128, 128), lambda j, i: (0, j)),
        pl.BlockSpec((128, 128), lambda j, i: (0, j)),
        pl.BlockSpec((1, 4, 32), lambda j, i: (j, 0, 0)),
        pl.BlockSpec((1, 4, 32), lambda j, i: (j, 0, 0)),
        pl.BlockSpec((1, 4, 32), lambda j, i: (j, 0, 0)),
        pl.BlockSpec((1, 4, 32), lambda j, i: (j, 0, 0)),
    ],
    out_specs=[
        pl.BlockSpec((1, BN, 128), lambda j, i: (j, i, 0)),
        pl.BlockSpec((1, BN, 128), lambda j, i: (j, i, 0)),
        pl.BlockSpec((1, BN, 4), lambda j, i: (j, i, 0)),
        pl.BlockSpec((1, BN, 4), lambda j, i: (j, i, 0)),
        pl.BlockSpec((1, 8, 4), lambda j, i: (j, 0, 0)),
    ],
    out_shape=[
        jax.ShapeDtypeStruct((2, NPAD, 128), _f32),
        jax.ShapeDtypeStruct((2, NPAD, 128), _f32),
        jax.ShapeDtypeStruct((2, NPAD, 4), _f32),
        jax.ShapeDtypeStruct((2, NPAD, 4), _f32),
        jax.ShapeDtypeStruct((2, 8, 4), _f32),
    ],
)


def _make_ka():

    def body(sstbl_s, sstbl_t, srcl, trgl, epl, cvec, w_out, dpart,
             srcidx_v, trgidx_v, ep_v, cv_v, srows_v, trows_v, wbuf_v,
             dsp, sem):
        c = lax.axis_index("c")
        s = lax.axis_index("s")
        wid = s * NC + c

        def _zr(i, _):
            wbuf_v[i] = jnp.zeros((16,), _f32)
            return 0
        lax.fori_loop(0, CA, _zr, 0)
        for q in range(RPT // CA):
            pltpu.sync_copy(wbuf_v, dsp.at[pl.ds(s * RPT + q * CA, CA)])
        plsc.subcore_barrier()

        pltpu.sync_copy(cvec, cv_v)
        cv = cv_v[...]
        iot = lax.iota(jnp.int32, 16)
        lane8 = iot < 8

        def _chunk(k, _):
            base = wid * EW_A + k * CA
            pltpu.sync_copy(srcl.at[pl.ds(base, CA)], srcidx_v)
            pltpu.sync_copy(trgl.at[pl.ds(base, CA)], trgidx_v)
            pltpu.sync_copy(epl.at[pl.ds(base, CA)], ep_v.at[pl.ds(0, CA)])
            pltpu.async_copy(sstbl_s.at[srcidx_v], srows_v, sem).wait()
            pltpu.async_copy(sstbl_t.at[trgidx_v], trows_v, sem).wait()

            def _edge(e, _):
                pe = ep_v[pl.ds(e, 16)][0]
                sc = (srows_v[e, pl.ds(0, 16)] + trows_v[e, pl.ds(0, 16)]
                      + pe * cv)
                sc = jnp.maximum(sc, 0.2 * sc)
                w = jnp.where(lane8, jnp.exp(sc), 0.0)
                wbuf_v[e] = w
                return 0
            lax.fori_loop(0, CA, _edge, 0, unroll=8)
            pltpu.sync_copy(wbuf_v, w_out.at[pl.ds(base, CA)])
            pltpu.sync_copy(wbuf_v, dsp.at[trgidx_v], add=True)
            return 0
        lax.fori_loop(0, EW_A // CA, _chunk, 0)

        plsc.subcore_barrier()
        for q in range(RPT // CA):
            r0 = s * RPT + q * CA
            pltpu.sync_copy(dsp.at[pl.ds(r0, CA)],
                            dpart.at[pl.ds(c * NPAD + r0, CA)])

    return pl.kernel(
        body,
        out_type=[
            jax.ShapeDtypeStruct((E, 16), _f32),
            jax.ShapeDtypeStruct((NC * NPAD, 16), _f32),
        ],
        mesh=_mesh,
        scratch_types=[
            pltpu.VMEM((CA,), jnp.int32),
            pltpu.VMEM((CA,), jnp.int32),
            pltpu.VMEM((CA + 16,), _f32),
            pltpu.VMEM((16,), _f32),
            pltpu.VMEM((CA, 128), _f32),
            pltpu.VMEM((CA, 128), _f32),
            pltpu.VMEM((CA, 16), _f32),
            pltpu.VMEM_SHARED((NPAD, 16), _f32),
            pltpu.SemaphoreType.DMA,
        ],
    )


_ka = _make_ka()


def _make_spmm(mode):
    NCH = EW_B // CB

    def body(table, gidxl, sidxl, scalel, out2,
             g0, s0, w0, r0, g1, s1, w1, r1, acc_sp, sem0, sem1):
        c = lax.axis_index("c")
        s = lax.axis_index("s")
        slots = ((g0, s0, w0, r0, sem0), (g1, s1, w1, r1, sem1))

        def _zr(i, _):
            for k2 in range(8):
                r0[i, pl.ds(16 * k2, 16)] = jnp.zeros((16,), _f32)
            return 0
        lax.fori_loop(0, CB, _zr, 0)
        for q in range(RPT // CB):
            pltpu.sync_copy(r0, acc_sp.at[pl.ds(s * RPT + q * CB, CB)])
        plsc.subcore_barrier()

        coff = c * NPAD

        def _stage1(k, slot):
            gv, sv, wv, rv, sem = slot
            base = s * EW_B + k * CB
            pltpu.sync_copy(gidxl.at[pl.ds(base, CB)], gv)
            pltpu.sync_copy(sidxl.at[pl.ds(base, CB)], sv)
            if mode == "gat":
                pltpu.sync_copy(scalel.at[pl.ds(base, CB)], wv)
            else:
                pltpu.sync_copy(scalel.at[pl.ds(base, CB)],
                                wv.at[pl.ds(0, CB)])
            for i in range(CB // 16):
                gv[pl.ds(16 * i, 16)] = gv[pl.ds(16 * i, 16)] + coff
            pltpu.async_copy(table.at[gv], rv, sem)

        def _stage2(slot):
            gv, sv, wv, rv, sem = slot
            pltpu.make_async_copy(table.at[gv], rv, sem).wait()
            if mode == "gat":
                lbase = 4 * c

                @plsc.parallel_loop(0, CB, step=1, unroll=8)
                def _edge(e):
                    wrow = wv[e]
                    for m in range(4):
                        idxv = jnp.zeros((16,), jnp.int32) + (lbase + m)
                        b = wrow.at[idxv].get(mode="promise_in_bounds")
                        for k2 in (2 * m, 2 * m + 1):
                            sl = pl.ds(16 * k2, 16)
                            rv[e, sl] = rv[e, sl] * b
            else:

                @plsc.parallel_loop(0, CB, step=1, unroll=8)
                def _edge(e):
                    pe = wv[pl.ds(e, 16)][0]
                    for k2 in range(8):
                        sl = pl.ds(16 * k2, 16)
                        rv[e, sl] = rv[e, sl] * pe
            pltpu.sync_copy(rv, acc_sp.at[sv], add=True)

        _stage1(0, slots[0])

        @pl.loop(0, NCH, step=2)
        def _pair(g):
            _stage1(g + 1, slots[1])
            _stage2(slots[0])

            @pl.when(g + 2 < NCH)
            def _():
                _stage1(g + 2, slots[0])
            _stage2(slots[1])

        plsc.subcore_barrier()
        for q in range(RPT // CB):
            rq = s * RPT + q * CB
            pltpu.sync_copy(acc_sp.at[pl.ds(rq, CB)],
                            out2.at[pl.ds(coff + rq, CB)])

    scale_shape = (CB, 16) if mode == "gat" else (CB + 16,)
    return pl.kernel(
        body,
        out_type=jax.ShapeDtypeStruct((NC * NPAD, 128), _f32),
        mesh=_mesh,
        scratch_types=[
            pltpu.VMEM((CB,), jnp.int32),
            pltpu.VMEM((CB,), jnp.int32),
            pltpu.VMEM(scale_shape, _f32),
            pltpu.VMEM((CB, 128), _f32),
            pltpu.VMEM((CB,), jnp.int32),
            pltpu.VMEM((CB,), jnp.int32),
            pltpu.VMEM(scale_shape, _f32),
            pltpu.VMEM((CB, 128), _f32),
            pltpu.VMEM_SHARED((NPAD, 128), _f32),
            pltpu.SemaphoreType.DMA,
            pltpu.SemaphoreType.DMA,
        ],
    )


_kb_gat = _make_spmm("gat")
_kb_hop = _make_spmm("hop")


def _k4_body(num_ref, d0_ref, d1_ref, skip_ref, bias_ref, x0_ref):
    j = pl.program_id(0)
    d = d0_ref[0] + d1_ref[0]
    d4 = jnp.where(j == 0, d[:, 0:4], d[:, 4:8])
    d128 = jnp.broadcast_to(d4[:, :, None], (BN, 4, 32)).reshape(BN, 128)
    o = num_ref[0] / (d128 + 1e-16) + skip_ref[0] + bias_ref[0]
    x0_ref[0] = jnp.where(o > 0, o, jnp.exp(jnp.minimum(o, 0.0)) - 1.0)


_k4 = pl.pallas_call(
    _k4_body,
    grid=(2, NB),
    in_specs=[
        pl.BlockSpec((1, BN, 128), lambda j, i: (j, i, 0)),
        pl.BlockSpec((1, BN, 16), lambda j, i: (0, i, 0)),
        pl.BlockSpec((1, BN, 16), lambda j, i: (1, i, 0)),
        pl.BlockSpec((1, BN, 128), lambda j, i: (j, i, 0)),
        pl.BlockSpec((1, 1, 128), lambda j, i: (j, 0, 0)),
    ],
    out_specs=pl.BlockSpec((1, BN, 128), lambda j, i: (j, i, 0)),
    out_shape=jax.ShapeDtypeStruct((2, NPAD, 128), _f32),
)


def _sig(v):
    return 1.0 / (1.0 + jnp.exp(-v))


def _k6_body(x00, x01, x10, x11, x20, x21, x30, x31,
             ipw_ref, cwt_ref, xpw_ref, dtw_ref, alt_ref, mow_ref, wout_ref,
             vecs_ref, emb_ref):
    xs = [jnp.concatenate([a[0], b[0]], axis=1)
          for a, b in ((x00, x01), (x10, x11), (x20, x21), (x30, x31))]
    conv_b = vecs_ref[0:1]
    dt_b = vecs_ref[1:2]
    d_param = vecs_ref[2:3]
    ln_g = vecs_ref[3:4]
    ln_b = vecs_ref[4:5]
    b_out = vecs_ref[5:6]

    xm = []
    z3 = None
    for t in range(4):
        xz = jnp.dot(xs[t], ipw_ref[...], preferred_element_type=_f32)
        xm.append(xz[:, :DI])
        if t == 3:
            z3 = xz[:, DI:]
    cwt = cwt_ref[...]
    xmc = []
    for t in range(4):
        acc = jnp.broadcast_to(conv_b, (BN, DI))
        for jj in range(t + 1):
            acc = acc + xm[jj] * cwt[jj + 3 - t]
        xmc.append(acc * _sig(acc))
    A = -jnp.exp(alt_ref[...])
    hs = [jnp.zeros((BN, DI), _f32) for _ in range(DS)]
    y3 = None
    for t in range(4):
        xd = jnp.dot(xmc[t], xpw_ref[...], preferred_element_type=_f32)
        dtv = jnp.dot(xd[:, :DTR], dtw_ref[...],
                      preferred_element_type=_f32) + dt_b
        dtv = jnp.maximum(dtv, 0.0) + jnp.log(1.0 + jnp.exp(-jnp.abs(dtv)))
        u = dtv * xmc[t]
        Bm = xd[:, DTR:DTR + DS]
        for s2 in range(DS):
            hs[s2] = hs[s2] * jnp.exp(dtv * A[s2]) + u * Bm[:, s2:s2 + 1]
        if t == 3:
            C3 = xd[:, DTR + DS:DTR + 2 * DS]
            y3 = hs[0] * C3[:, 0:1]
            for s2 in range(1, DS):
                y3 = y3 + hs[s2] * C3[:, s2:s2 + 1]
    y = y3 + xmc[3] * d_param
    y = y * (z3 * _sig(z3))
    o3 = jnp.dot(y, mow_ref[...], preferred_element_type=_f32)
    v = o3 + xs[0]
    mu = jnp.mean(v, axis=1, keepdims=True)
    var = jnp.mean((v - mu) ** 2, axis=1, keepdims=True)
    lnv = (v - mu) / jnp.sqrt(var + 1e-5) * ln_g + ln_b
    emb_ref[...] = jnp.dot(lnv, wout_ref[...], preferred_element_type=_f32) + b_out


def _xspec(h):
    return pl.BlockSpec((1, BN, 128), lambda i, _h=h: (_h, i, 0))


_k6 = pl.pallas_call(
    _k6_body,
    grid=(NB,),
    in_specs=(
        [_xspec(h) for _t in range(4) for h in (0, 1)]
        + [
            pl.BlockSpec((DM, 2 * DI), lambda i: (0, 0)),
            pl.BlockSpec((4, 1, DI), lambda i: (0, 0, 0)),
            pl.BlockSpec((DI, DTR + 2 * DS), lambda i: (0, 0)),
            pl.BlockSpec((DTR, DI), lambda i: (0, 0)),
            pl.BlockSpec((DS, DI), lambda i: (0, 0)),
            pl.BlockSpec((DI, DM), lambda i: (0, 0)),
            pl.BlockSpec((DM, DM), lambda i: (0, 0)),
            pl.BlockSpec((8, DM), lambda i: (0, 0)),
        ]
    ),
    out_specs=pl.BlockSpec((BN, DM), lambda i: (i, 0)),
    out_shape=jax.ShapeDtypeStruct((NPAD, DM), _f32),
)


def _k7_body(emb, idxl, out, idx_v, rows_v, sem):
    c = lax.axis_index("c")
    s = lax.axis_index("s")
    wid = s * NC + c
    for q in range(2):
        base = wid * (BSEQ * TSEQ // NW) + q * 128
        pltpu.sync_copy(idxl.at[pl.ds(base, 128)], idx_v)
        pltpu.async_copy(emb.at[idx_v], rows_v, sem).wait()
        pltpu.sync_copy(rows_v, out.at[pl.ds(base, 128)])


_k7 = pl.kernel(
    _k7_body,
    out_type=jax.ShapeDtypeStruct((BSEQ * TSEQ, DM), _f32),
    mesh=_mesh,
    scratch_types=[
        pltpu.VMEM((128,), jnp.int32),
        pltpu.VMEM((128, DM), _f32),
        pltpu.SemaphoreType.DMA,
    ],
)


def kernel(node_features, edge_index_input, edge_prob_input, x, W_proj, W_prob,
           a_src, a_trg, a_prob, W_skip, gat_bias, in_proj_W, conv_w, conv_b,
           x_proj_W, dt_proj_W, dt_proj_b, A_log, D_param, mamba_out_W,
           ln_g, ln_b, W_out, b_out):
    nfp = jnp.pad(node_features, ((0, NPAD - N), (0, 0)))
    src = edge_index_input[0]
    trg = edge_index_input[1]
    ep = edge_prob_input.reshape(E)

    proj2, skip2, ss_o, st_o, cv_o = _k1(
        nfp, W_proj, W_skip,
        a_src.reshape(2, 4, 32), a_trg.reshape(2, 4, 32),
        W_prob.reshape(2, 4, 32), a_prob.reshape(2, 4, 32))

    z120 = jnp.zeros((NPAD, 120), _f32)
    sstbl_s = jnp.concatenate([ss_o[0], ss_o[1], z120], axis=1)
    sstbl_t = jnp.concatenate([st_o[0], st_o[1], z120], axis=1)
    cvec16 = jnp.concatenate([cv_o[0, 0], cv_o[1, 0], jnp.zeros((8,), _f32)])

    w_e, dpart = _ka(sstbl_s, sstbl_t, src, trg, ep, cvec16)

    num2 = _kb_gat(proj2.reshape(NC * NPAD, 128), src, trg, w_e)

    x0_2 = _k4(num2.reshape(2, NPAD, 128), dpart.reshape(2, NPAD, 16),
               dpart.reshape(2, NPAD, 16), skip2, gat_bias.reshape(2, 1, 128))

    x0f = x0_2.reshape(NC * NPAD, 128)
    x1f = _kb_hop(x0f, trg, src, ep)
    x2f = _kb_hop(x1f, trg, src, ep)
    x3f = _kb_hop(x2f, trg, src, ep)

    vecs8 = jnp.stack([conv_b, dt_proj_b, D_param, ln_g, ln_b, b_out,
                       jnp.zeros((DM,), _f32), jnp.zeros((DM,), _f32)])
    xr = [v.reshape(2, NPAD, 128) for v in (x0f, x1f, x2f, x3f)]
    emb = _k6(xr[0], xr[0], xr[1], xr[1], xr[2], xr[2], xr[3], xr[3],
              in_proj_W, conv_w.T.reshape(4, 1, DI), x_proj_W, dt_proj_W,
              A_log.T, mamba_out_W, W_out, vecs8)

    out = _k7(emb, x.reshape(-1))
    return out.reshape(BSEQ, TSEQ, DM)
